# Initial kernel scaffold; baseline (speedup 1.0000x reference)
import jax
import jax.numpy as jnp
from jax.experimental import pallas as pl


def kernel(categorical_features, embedding_table, proj_W, proj_b):
    raise NotImplementedError("write your pallas kernel here")


# speedup vs baseline: 22.2960x; 22.2960x over previous
import functools

import jax
import jax.numpy as jnp
import numpy as np
from jax.experimental import pallas as pl
from jax.experimental.pallas import tpu as pltpu
from jax.experimental.pallas import tpu_sc as plsc

_NUM_FEATURES = 26
_EMBED_DIM = 32
_OUTPUT_DIM = 128
_VOCAB_PER_FEATURE = 100000

_WINDOW_TOK = 64
_W = _WINDOW_TOK * _NUM_FEATURES
_LANES = 16

_BT = 1024


def _sc_gather(idx_flat, offs_tiled, table):
    n = idx_flat.shape[1]
    mesh = plsc.VectorSubcoreMesh(core_axis_name="core", subcore_axis_name="subcore")

    @functools.partial(
        pl.kernel,
        out_type=jax.ShapeDtypeStruct((n, _EMBED_DIM), jnp.float32),
        mesh=mesh,
        compiler_params=pltpu.CompilerParams(use_tc_tiling_on_sc=False),
    )
    def k(idx_hbm, off_hbm, tab_hbm, out_hbm):
        def body(idx_vmem, off_vmem, out_vmem):
            @pl.loop(0, _W, step=_LANES)
            def _(j):
                slc = (pl.ds(0, 1), pl.ds(j, _LANES))
                idx_vmem.at[*slc][...] = (
                    idx_vmem.at[*slc][...] + off_vmem.at[*slc][...]
                )

            pltpu.sync_copy(tab_hbm.at[idx_vmem.at[0]], out_vmem)

        pltpu.emit_pipeline(
            body,
            grid=(n // _W,),
            in_specs=[
                pl.BlockSpec((1, _W), lambda i: (0, i)),
                pl.BlockSpec((1, _W), lambda i: (0, 0)),
            ],
            out_specs=[pl.BlockSpec((_W, _EMBED_DIM), lambda i: (i, 0))],
            core_axis_name=("core", "subcore"),
            dimension_semantics=(pltpu.PARALLEL,),
        )(idx_hbm, off_hbm, out_hbm)

    return k(idx_flat, offs_tiled, table)


def _tc_project(emb, w, b2d):
    t = emb.shape[0]
    fd = emb.shape[1]

    def body(e_ref, w_ref, b_ref, o_ref):
        o_ref[...] = (
            jnp.dot(e_ref[...], w_ref[...], preferred_element_type=jnp.float32)
            + b_ref[...]
        )

    return pl.pallas_call(
        body,
        grid=(t // _BT,),
        in_specs=[
            pl.BlockSpec((_BT, fd), lambda i: (i, 0)),
            pl.BlockSpec((fd, _OUTPUT_DIM), lambda i: (0, 0)),
            pl.BlockSpec((1, _OUTPUT_DIM), lambda i: (0, 0)),
        ],
        out_specs=pl.BlockSpec((_BT, _OUTPUT_DIM), lambda i: (i, 0)),
        out_shape=jax.ShapeDtypeStruct((t, _OUTPUT_DIM), jnp.float32),
    )(emb, w, b2d)


def kernel(categorical_features, embedding_table, proj_W, proj_b):
    b, l, f = categorical_features.shape
    n = b * l * f
    idx = categorical_features.astype(jnp.int32).reshape(1, n)
    offs = jnp.asarray(
        np.tile(np.arange(_NUM_FEATURES, dtype=np.int32) * _VOCAB_PER_FEATURE,
                _WINDOW_TOK).reshape(1, _W)
    )
    emb = _sc_gather(idx, offs, embedding_table)
    emb2 = emb.reshape(b * l, f * _EMBED_DIM)
    out = _tc_project(emb2, proj_W, proj_b.reshape(1, _OUTPUT_DIM))
    return out.reshape(b, l, _OUTPUT_DIM)
